# baseline (speedup 1.0000x reference)
import functools

import jax
import jax.numpy as jnp
from jax import lax
from jax.experimental import pallas as pl
from jax.experimental.pallas import tpu as pltpu
from jax.experimental.pallas import tpu_sc as plsc

_B = 1024
_IJ = 26 * 20
_SRC = 64
_DST = 32
_LANE = 128
_XP = _IJ * _SRC * (_B // _LANE)
_QP = _IJ * _DST * (_B // _LANE)
_NW = 32
_PPW = _QP // _NW
_CHUNK = 80
_NCHUNK = _PPW // _CHUNK
_NVEC = _CHUNK // 16


@functools.partial(
    pl.kernel,
    out_type=jax.ShapeDtypeStruct((_QP, _LANE), jnp.float32),
    mesh=plsc.VectorSubcoreMesh(core_axis_name="c", subcore_axis_name="s"),
    compiler_params=pltpu.CompilerParams(needs_layout_passes=False),
    scratch_types=[
        pltpu.VMEM((_DST,), jnp.int32),
        pltpu.VMEM((_NCHUNK, _CHUNK), jnp.int32),
        pltpu.VMEM((2 * _CHUNK, _LANE), jnp.float32),
        pltpu.VMEM((2 * _CHUNK, _LANE), jnp.float32),
        pltpu.VMEM((2 * _CHUNK, _LANE), jnp.float32),
        pltpu.VMEM((2 * _CHUNK, _LANE), jnp.float32),
        pltpu.SemaphoreType.DMA,
        pltpu.SemaphoreType.DMA,
        pltpu.SemaphoreType.DMA,
        pltpu.SemaphoreType.DMA,
        pltpu.SemaphoreType.DMA,
        pltpu.SemaphoreType.DMA,
        pltpu.SemaphoreType.DMA,
        pltpu.SemaphoreType.DMA,
    ],
)
def _index_select_sc(xp_hbm, idx_hbm, out_hbm, idxtab_v, idxbuf,
                     buf0, buf1, buf2, buf3,
                     g0, g1, g2, g3, w0, w1, w2, w3):
    wid = lax.axis_index("s") * 2 + lax.axis_index("c")
    qbase = wid * _PPW

    pltpu.sync_copy(idx_hbm, idxtab_v)
    iota = lax.iota(jnp.int32, 16)

    buf = (buf0, buf1, buf2, buf3)
    gsem = (g0, g1, g2, g3)
    wsem = (w0, w1, w2, w3)

    def fill_indices(t):
        q0 = qbase + t * _CHUNK
        for v in range(_NVEC):
            q = q0 + v * 16 + iota
            ij = q >> 8
            r1 = q & 255
            k = ((r1 >> 6) << 3) | (r1 & 7)
            tc = (r1 >> 3) & 7
            c = plsc.load_gather(idxtab_v, [k])
            idxbuf[t, pl.ds(v * 16, 16)] = (
                (ij << 9) + ((c >> 3) << 6) + (tc << 3) + (c & 7)
            )

    npairs = _NCHUNK // 2

    def start_gather(t):
        b = (t // 2) % 4
        half = buf[b].at[pl.ds((t % 2) * _CHUNK, _CHUNK)]
        return pltpu.async_copy(xp_hbm.at[idxbuf.at[t]], half, gsem[b])

    def start_write(p):
        b = p % 4
        q0 = qbase + 2 * p * _CHUNK
        return pltpu.async_copy(buf[b], out_hbm.at[pl.ds(q0, 2 * _CHUNK)], wsem[b])

    gathers, writes = {}, {}
    for p in range(npairs):
        if p >= 4:
            writes[p - 4].wait()
        for t in (2 * p, 2 * p + 1):
            fill_indices(t)
            gathers[t] = start_gather(t)
        if p >= 3:
            gathers[2 * p - 6].wait()
            gathers[2 * p - 5].wait()
            writes[p - 3] = start_write(p - 3)
    for p in range(npairs - 3, npairs):
        gathers[2 * p].wait()
        gathers[2 * p + 1].wait()
        writes[p] = start_write(p)
    for p in range(npairs - 4, npairs):
        writes[p].wait()


def kernel(x, dim, index, out):
    axis = x.ndim - 1
    idx = (index + (dim - axis)).astype(jnp.int32)
    xp = (
        x.reshape(8, _LANE, 26, 20, 8, 8)
        .transpose(2, 3, 4, 0, 5, 1)
        .reshape(_XP, _LANE)
    )
    res = _index_select_sc(xp, idx)
    return (
        res.reshape(26, 20, 4, 8, 8, _LANE)
        .transpose(3, 5, 0, 1, 2, 4)
        .reshape(_B, 26, 20, _DST)
    )


# speedup vs baseline: 1.0934x; 1.0934x over previous
import functools

import jax
import jax.numpy as jnp
from jax import lax
from jax.experimental import pallas as pl
from jax.experimental.pallas import tpu as pltpu
from jax.experimental.pallas import tpu_sc as plsc

_B = 1024
_IJ = 26 * 20
_SRC = 64
_DST = 32
_LANE = 128
_XP = _IJ * _SRC * (_B // _LANE)
_QP = _IJ * _DST * (_B // _LANE)
_NW = 32
_PPW = _QP // _NW
_CHUNK = 80
_NCHUNK = _PPW // _CHUNK
_NVEC = _CHUNK // 16


@functools.partial(
    pl.kernel,
    out_type=jax.ShapeDtypeStruct((_QP, _LANE), jnp.float32),
    mesh=plsc.VectorSubcoreMesh(core_axis_name="c", subcore_axis_name="s"),
    compiler_params=pltpu.CompilerParams(needs_layout_passes=False),
    scratch_types=[
        pltpu.VMEM((_DST,), jnp.int32),
        pltpu.VMEM((_NCHUNK, _CHUNK), jnp.int32),
        pltpu.VMEM((_CHUNK, _LANE), jnp.float32),
        pltpu.VMEM((_CHUNK, _LANE), jnp.float32),
        pltpu.VMEM((_CHUNK, _LANE), jnp.float32),
        pltpu.VMEM((_CHUNK, _LANE), jnp.float32),
        pltpu.SemaphoreType.DMA,
        pltpu.SemaphoreType.DMA,
        pltpu.SemaphoreType.DMA,
        pltpu.SemaphoreType.DMA,
        pltpu.SemaphoreType.DMA,
        pltpu.SemaphoreType.DMA,
        pltpu.SemaphoreType.DMA,
        pltpu.SemaphoreType.DMA,
    ],
)
def _index_select_sc(xp_hbm, idx_hbm, out_hbm, idxtab_v, idxbuf,
                     buf0, buf1, buf2, buf3,
                     g0, g1, g2, g3, w0, w1, w2, w3):
    wid = lax.axis_index("s") * 2 + lax.axis_index("c")
    qbase = wid * _PPW

    pltpu.sync_copy(idx_hbm, idxtab_v)
    iota = lax.iota(jnp.int32, 16)

    buf = (buf0, buf1, buf2, buf3)
    gsem = (g0, g1, g2, g3)
    wsem = (w0, w1, w2, w3)

    def fill_indices(t):
        q0 = qbase + t * _CHUNK
        for v in range(_NVEC):
            q = q0 + v * 16 + iota
            ij = q >> 8
            r1 = q & 255
            k = ((r1 >> 6) << 3) | (r1 & 7)
            tc = (r1 >> 3) & 7
            c = plsc.load_gather(idxtab_v, [k])
            idxbuf[t, pl.ds(v * 16, 16)] = (
                (ij << 9) + ((c >> 3) << 6) + (tc << 3) + (c & 7)
            )

    def start_gather(t, b):
        return pltpu.async_copy(xp_hbm.at[idxbuf.at[t]], buf[b], gsem[b])

    def wait_gather(t, b):
        pltpu.make_async_copy(xp_hbm.at[idxbuf.at[t]], buf[b], gsem[b]).wait()

    def start_write(t, b):
        q0 = qbase + t * _CHUNK
        return pltpu.async_copy(buf[b], out_hbm.at[pl.ds(q0, _CHUNK)], wsem[b])

    def wait_write(t, b):
        q0 = qbase + t * _CHUNK
        pltpu.make_async_copy(buf[b], out_hbm.at[pl.ds(q0, _CHUNK)], wsem[b]).wait()

    @pl.loop(0, _NCHUNK // 4)
    def _steady(g):
        for j in range(4):
            t = g * 4 + j
            fill_indices(t)
            @pl.when(g > 0)
            def _():
                wait_write(t - 4, j)
            start_gather(t, j)
            bj = (j - 2) % 4
            if j >= 2:
                wait_gather(t - 2, bj)
                start_write(t - 2, bj)
            else:
                @pl.when(g > 0)
                def _():
                    wait_gather(t - 2, bj)
                    start_write(t - 2, bj)

    for t in range(_NCHUNK - 2, _NCHUNK):
        wait_gather(t, t % 4)
        start_write(t, t % 4)
    for t in range(_NCHUNK - 4, _NCHUNK):
        wait_write(t, t % 4)


def kernel(x, dim, index, out):
    axis = x.ndim - 1
    idx = (index + (dim - axis)).astype(jnp.int32)
    xp = (
        x.reshape(8, _LANE, 26, 20, 8, 8)
        .transpose(2, 3, 4, 0, 5, 1)
        .reshape(_XP, _LANE)
    )
    res = _index_select_sc(xp, idx)
    return (
        res.reshape(26, 20, 4, 8, 8, _LANE)
        .transpose(3, 5, 0, 1, 2, 4)
        .reshape(_B, 26, 20, _DST)
    )
